# Initial kernel scaffold; baseline (speedup 1.0000x reference)
import jax
import jax.numpy as jnp
from jax.experimental import pallas as pl


def kernel(input_ids, embed_table):
    raise NotImplementedError("write your pallas kernel here")


# speedup vs baseline: 1.8377x; 1.8377x over previous
import functools

import jax
import jax.numpy as jnp
from jax import lax
from jax.experimental import pallas as pl
from jax.experimental.pallas import tpu as pltpu
from jax.experimental.pallas import tpu_sc as plsc

_NUM_CORES = 2
_NUM_SUBCORES = 16
_NUM_WORKERS = _NUM_CORES * _NUM_SUBCORES

_CHUNK = 16


@functools.lru_cache(maxsize=None)
def _build(num_ids: int, d_model: int):
    assert num_ids % (_NUM_WORKERS * _CHUNK) == 0
    ids_per_worker = num_ids // _NUM_WORKERS
    n_chunks = ids_per_worker // _CHUNK

    mesh = plsc.VectorSubcoreMesh(core_axis_name="c", subcore_axis_name="s")

    @functools.partial(
        pl.kernel,
        mesh=mesh,
        out_type=jax.ShapeDtypeStruct((num_ids, d_model), jnp.float32),
        scratch_types=[
            pltpu.VMEM((ids_per_worker,), jnp.int32),
            pltpu.VMEM((_CHUNK, d_model), jnp.float32),
            pltpu.VMEM((_CHUNK, d_model), jnp.float32),
            pltpu.SemaphoreType.DMA,
            pltpu.SemaphoreType.DMA,
        ],
    )
    def gather_kernel(ids_hbm, table_hbm, out_hbm, idx_v, buf0, buf1, sem0, sem1):
        wid = lax.axis_index("s") * _NUM_CORES + lax.axis_index("c")
        base = wid * ids_per_worker
        pltpu.sync_copy(ids_hbm.at[pl.ds(base, ids_per_worker)], idx_v)

        bufs = (buf0, buf1)
        sems = (sem0, sem1)

        def gather(i, b):
            return pltpu.make_async_copy(
                table_hbm.at[idx_v.at[pl.ds(i * _CHUNK, _CHUNK)]],
                bufs[b],
                sems[b],
            )

        gather(0, 0).start()
        gather(1, 1).start()

        def body(t, carry):
            for b in range(2):
                i = 2 * t + b
                gather(i, b).wait()
                pltpu.sync_copy(
                    bufs[b], out_hbm.at[pl.ds(base + i * _CHUNK, _CHUNK)]
                )
                nxt = i + 2

                @pl.when(nxt < n_chunks)
                def _():
                    gather(nxt, b).start()

            return carry

        lax.fori_loop(0, n_chunks // 2, body, 0, unroll=False)

    return gather_kernel


def kernel(input_ids, embed_table):
    num_ids = input_ids.shape[0] * input_ids.shape[1]
    d_model = embed_table.shape[1]
    ids = input_ids.reshape(num_ids).astype(jnp.int32)
    out = _build(num_ids, d_model)(ids, embed_table)
    return out.reshape(input_ids.shape + (d_model,))


# speedup vs baseline: 1.8430x; 1.0028x over previous
import functools

import jax
import jax.numpy as jnp
from jax import lax
from jax.experimental import pallas as pl
from jax.experimental.pallas import tpu as pltpu
from jax.experimental.pallas import tpu_sc as plsc

_NUM_CORES = 2
_NUM_SUBCORES = 16
_NUM_WORKERS = _NUM_CORES * _NUM_SUBCORES

_CHUNK = 16


@functools.lru_cache(maxsize=None)
def _build(num_ids: int, d_model: int):
    assert num_ids % (_NUM_WORKERS * _CHUNK) == 0
    ids_per_worker = num_ids // _NUM_WORKERS
    n_chunks = ids_per_worker // _CHUNK

    mesh = plsc.VectorSubcoreMesh(core_axis_name="c", subcore_axis_name="s")

    @functools.partial(
        pl.kernel,
        mesh=mesh,
        out_type=jax.ShapeDtypeStruct((num_ids, d_model), jnp.float32),
        scratch_types=[
            pltpu.VMEM((ids_per_worker,), jnp.int32),
            pltpu.VMEM((_CHUNK, d_model), jnp.float32),
            pltpu.VMEM((_CHUNK, d_model), jnp.float32),
            pltpu.VMEM((_CHUNK, d_model), jnp.float32),
            pltpu.SemaphoreType.DMA,
            pltpu.SemaphoreType.DMA,
            pltpu.SemaphoreType.DMA,
        ],
    )
    def gather_kernel(
        ids_hbm, table_hbm, out_hbm, idx_v, buf0, buf1, buf2, sem0, sem1, sem2
    ):
        wid = lax.axis_index("s") * _NUM_CORES + lax.axis_index("c")
        base = wid * ids_per_worker
        pltpu.sync_copy(ids_hbm.at[pl.ds(base, ids_per_worker)], idx_v)

        bufs = (buf0, buf1, buf2)
        sems = (sem0, sem1, sem2)
        nbuf = len(bufs)

        def gather(i, b):
            return pltpu.make_async_copy(
                table_hbm.at[idx_v.at[pl.ds(i * _CHUNK, _CHUNK)]],
                bufs[b],
                sems[b],
            )

        for b in range(nbuf):
            gather(b, b).start()

        n_steady = (n_chunks - nbuf) // nbuf

        def body(t, carry):
            for b in range(nbuf):
                i = nbuf * t + b
                gather(i, b).wait()
                pltpu.sync_copy(
                    bufs[b], out_hbm.at[pl.ds(base + i * _CHUNK, _CHUNK)]
                )
                gather(i + nbuf, b).start()
            return carry

        lax.fori_loop(0, n_steady, body, 0, unroll=False)

        for i in range(n_steady * nbuf, n_chunks):
            b = i % nbuf
            nxt = i + nbuf
            gather(i, b).wait()
            pltpu.sync_copy(
                bufs[b], out_hbm.at[pl.ds(base + i * _CHUNK, _CHUNK)]
            )
            if (n_steady + 1) * nbuf <= nxt < n_chunks:
                gather(nxt, b).start()

    return gather_kernel


def kernel(input_ids, embed_table):
    num_ids = input_ids.shape[0] * input_ids.shape[1]
    d_model = embed_table.shape[1]
    ids = input_ids.reshape(num_ids).astype(jnp.int32)
    out = _build(num_ids, d_model)(ids, embed_table)
    return out.reshape(input_ids.shape + (d_model,))
